# baseline (speedup 1.0000x reference)
import functools

import jax
import jax.numpy as jnp
from jax import lax
from jax.experimental import pallas as pl
from jax.experimental.pallas import tpu as pltpu, tpu_sc as plsc

N_FIELDS = 26
VOCAB = 100000
EMBED = 32
BATCH = 16384
NF_K = 13

_INFO = plsc.get_sparse_core_info()
_NC, _NS = _INFO.num_cores, _INFO.num_subcores
_NW = _NC * _NS
_BPW = BATCH // _NW
_NG = 4
_GSZ = _BPW // _NG


def _body(ids_hbm, tab_hbm, out_hbm, idx_v, e0_v, e1_v, t0, t1, t2, t3,
          isem, gsem0, gsem1, wsem0, wsem1, wsem2, wsem3):
    wid = lax.axis_index("s") * _NC + lax.axis_index("c")
    base = wid * _BPW

    def fire_ids(f, _):
        for g in range(_NG):
            pltpu.async_copy(
                ids_hbm.at[pl.ds(f * BATCH + base + g * _GSZ, _GSZ)],
                idx_v.at[f, g],
                isem,
            )
        return 0

    lax.fori_loop(0, NF_K, fire_ids, 0)

    def drain_ids(t, _):
        pltpu.make_async_copy(
            ids_hbm.at[pl.ds(0, _GSZ)], idx_v.at[0, 0], isem
        ).wait()
        return 0

    lax.fori_loop(0, NF_K * _NG, drain_ids, 0)

    def build_eidx(f, e_v):
        def per_eg(t, _):
            e = t // _NG
            g = t - e * _NG
            off = (f * EMBED + e) * VOCAB
            for u in range(_GSZ // 16):
                s16 = pl.ds(u * 16, 16)
                e_v[e, g, s16] = idx_v[f, g, s16] + off
            return 0

        lax.fori_loop(0, EMBED * _NG, per_eg, 0)

    def fire_field(e_v, t_v, gsem):
        def fire_e(e, _):
            for g in range(_NG):
                pltpu.async_copy(
                    tab_hbm.at[e_v.at[e, g]],
                    t_v.at[e, pl.ds(g * _GSZ, _GSZ)],
                    gsem,
                )
            return 0

        lax.fori_loop(0, EMBED, fire_e, 0)

    def drain_field(gsem):
        pltpu.make_async_copy(
            out_hbm.at[pl.ds(0, EMBED), pl.ds(0, _BPW)], t0, gsem
        ).wait()

    def reclaim(t_v, wsem):
        pltpu.make_async_copy(
            t_v, out_hbm.at[pl.ds(0, EMBED), pl.ds(base, _BPW)], wsem
        ).wait()

    def write_field(f, t_v, wsem):
        pltpu.async_copy(
            t_v, out_hbm.at[pl.ds(f * EMBED, EMBED), pl.ds(base, _BPW)], wsem
        )

    ts = (t0, t1, t2, t3)
    ws = (wsem0, wsem1, wsem2, wsem3)
    gs = (gsem0, gsem1)
    es = (e0_v, e1_v)

    build_eidx(0, e0_v)
    fire_field(e0_v, t0, gsem0)
    build_eidx(1, e1_v)
    fire_field(e1_v, t1, gsem1)

    def quad(q, _):
        f_base = 4 * q
        for k in range(4):
            f = f_base + k
            drain_field(gs[k % 2])

            @pl.when(f + 2 < NF_K)
            def _():
                build_eidx(f + 2, es[k % 2])

                @pl.when(f >= 2)
                def _():
                    reclaim(ts[(k + 2) % 4], ws[(k + 2) % 4])

                fire_field(es[k % 2], ts[(k + 2) % 4], gs[k % 2])

            write_field(f, ts[k], ws[k])
        return 0

    lax.fori_loop(0, NF_K // 4, quad, 0)

    drain_field(gsem0)
    write_field(NF_K - 1, t0, wsem0)

    for k in range(4):
        reclaim(ts[k], ws[k])


def _run_half(ids_half, tab_half):
    run = pl.kernel(
        _body,
        out_type=jax.ShapeDtypeStruct((NF_K * EMBED, BATCH), jnp.float32),
        mesh=plsc.VectorSubcoreMesh(core_axis_name="c", subcore_axis_name="s"),
        scratch_types=[
            pltpu.VMEM((NF_K, _NG, _GSZ), jnp.int32),
            pltpu.VMEM((EMBED, _NG, _GSZ), jnp.int32),
            pltpu.VMEM((EMBED, _NG, _GSZ), jnp.int32),
            pltpu.VMEM((EMBED, _BPW), jnp.float32),
            pltpu.VMEM((EMBED, _BPW), jnp.float32),
            pltpu.VMEM((EMBED, _BPW), jnp.float32),
            pltpu.VMEM((EMBED, _BPW), jnp.float32),
            pltpu.SemaphoreType.DMA,
            pltpu.SemaphoreType.DMA,
            pltpu.SemaphoreType.DMA,
            pltpu.SemaphoreType.DMA,
            pltpu.SemaphoreType.DMA,
            pltpu.SemaphoreType.DMA,
            pltpu.SemaphoreType.DMA,
        ],
        compiler_params=pltpu.CompilerParams(use_tc_tiling_on_sc=False),
    )
    return run(ids_half, tab_half)


@functools.partial(jax.jit, static_argnums=())
def kernel(ids, tables):
    outs = []
    for h in range(2):
        ids_h = ids[h * NF_K:(h + 1) * NF_K].reshape(NF_K * BATCH)
        tab_h = (tables[h * NF_K:(h + 1) * NF_K]
                 .transpose(0, 2, 1)
                 .reshape(NF_K * EMBED * VOCAB))
        outs.append(_run_half(ids_h, tab_h))
    return jnp.concatenate(outs, axis=0).T


# speedup vs baseline: 1.0529x; 1.0529x over previous
import functools

import jax
import jax.numpy as jnp
from jax import lax
from jax.experimental import pallas as pl
from jax.experimental.pallas import tpu as pltpu, tpu_sc as plsc

N_FIELDS = 26
VOCAB = 100000
EMBED = 32
BATCH = 16384

_INFO = plsc.get_sparse_core_info()
_NC, _NS = _INFO.num_cores, _INFO.num_subcores
_NW = _NC * _NS
_BPW = BATCH // _NW
_NG = 4
_GSZ = _BPW // _NG


def _body(ids_hbm, tab_hbm, out_hbm, idx_v, e0_v, e1_v, t0, t1, t2, t3,
          isem, gsem0, gsem1, wsem0, wsem1, wsem2, wsem3):
    wid = lax.axis_index("s") * _NC + lax.axis_index("c")
    base = wid * _BPW

    def fire_ids(f, _):
        for g in range(_NG):
            pltpu.async_copy(
                ids_hbm.at[pl.ds(f * BATCH + base + g * _GSZ, _GSZ)],
                idx_v.at[f, g],
                isem,
            )
        return 0

    lax.fori_loop(0, N_FIELDS, fire_ids, 0)

    def drain_ids(t, _):
        pltpu.make_async_copy(
            ids_hbm.at[pl.ds(0, _GSZ)], idx_v.at[0, 0], isem
        ).wait()
        return 0

    lax.fori_loop(0, N_FIELDS * _NG, drain_ids, 0)

    def build_eidx(f, e_v):
        def per_eg(t, _):
            e = t // _NG
            g = t - e * _NG
            off = (f * EMBED + e) * VOCAB
            for u in range(_GSZ // 16):
                s16 = pl.ds(u * 16, 16)
                e_v[e, g, s16] = idx_v[f, g, s16] + off
            return 0

        lax.fori_loop(0, EMBED * _NG, per_eg, 0)

    def fire_field(e_v, t_v, gsem):
        def fire_e(e, _):
            for g in range(_NG):
                pltpu.async_copy(
                    tab_hbm.at[e_v.at[e, g]],
                    t_v.at[e, pl.ds(g * _GSZ, _GSZ)],
                    gsem,
                )
            return 0

        lax.fori_loop(0, EMBED, fire_e, 0)

    def drain_field(gsem):
        pltpu.make_async_copy(
            out_hbm.at[pl.ds(0, EMBED), pl.ds(0, _BPW)], t0, gsem
        ).wait()

    def reclaim(t_v, wsem):
        pltpu.make_async_copy(
            t_v, out_hbm.at[pl.ds(0, EMBED), pl.ds(base, _BPW)], wsem
        ).wait()

    def write_field(f, t_v, wsem):
        pltpu.async_copy(
            t_v, out_hbm.at[pl.ds(f * EMBED, EMBED), pl.ds(base, _BPW)], wsem
        )

    ts = (t0, t1, t2, t3)
    ws = (wsem0, wsem1, wsem2, wsem3)
    gs = (gsem0, gsem1)
    es = (e0_v, e1_v)

    build_eidx(0, e0_v)
    fire_field(e0_v, t0, gsem0)
    build_eidx(1, e1_v)
    fire_field(e1_v, t1, gsem1)

    def quad(q, _):
        f_base = 4 * q
        for k in range(4):
            f = f_base + k
            drain_field(gs[k % 2])

            @pl.when(f + 2 < N_FIELDS)
            def _():
                build_eidx(f + 2, es[k % 2])

                @pl.when(f >= 2)
                def _():
                    reclaim(ts[(k + 2) % 4], ws[(k + 2) % 4])

                fire_field(es[k % 2], ts[(k + 2) % 4], gs[k % 2])

            write_field(f, ts[k], ws[k])
        return 0

    lax.fori_loop(0, N_FIELDS // 4, quad, 0)

    drain_field(gsem0)
    write_field(N_FIELDS - 2, t0, wsem0)
    drain_field(gsem1)
    write_field(N_FIELDS - 1, t1, wsem1)

    for k in range(4):
        reclaim(ts[k], ws[k])


@functools.partial(jax.jit, static_argnums=())
def kernel(ids, tables):
    ids_flat = ids.reshape(N_FIELDS * BATCH)
    tab = tables.transpose(0, 2, 1).reshape(N_FIELDS * EMBED * VOCAB)
    run = pl.kernel(
        _body,
        out_type=jax.ShapeDtypeStruct((N_FIELDS * EMBED, BATCH), jnp.float32),
        mesh=plsc.VectorSubcoreMesh(core_axis_name="c", subcore_axis_name="s"),
        scratch_types=[
            pltpu.VMEM((N_FIELDS, _NG, _GSZ), jnp.int32),
            pltpu.VMEM((EMBED, _NG, _GSZ), jnp.int32),
            pltpu.VMEM((EMBED, _NG, _GSZ), jnp.int32),
            pltpu.VMEM((EMBED, _BPW), jnp.float32),
            pltpu.VMEM((EMBED, _BPW), jnp.float32),
            pltpu.VMEM((EMBED, _BPW), jnp.float32),
            pltpu.VMEM((EMBED, _BPW), jnp.float32),
            pltpu.SemaphoreType.DMA,
            pltpu.SemaphoreType.DMA,
            pltpu.SemaphoreType.DMA,
            pltpu.SemaphoreType.DMA,
            pltpu.SemaphoreType.DMA,
            pltpu.SemaphoreType.DMA,
            pltpu.SemaphoreType.DMA,
        ],
        compiler_params=pltpu.CompilerParams(use_tc_tiling_on_sc=False),
    )
    return run(ids_flat, tab).T
